# Initial kernel scaffold; baseline (speedup 1.0000x reference)
import functools

import jax
import jax.numpy as jnp
from jax import lax
from jax.experimental import pallas as pl
from jax.experimental.pallas import tpu as pltpu
from jax.experimental.pallas import tpu_sc as plsc

N = 10000
E = 320000
D = 128
DH = 64
LANES = 16
EDGE_ROWS = E // 128
ROWS_PER_TILE = 160
PAD_ROWS = 16 * ROWS_PER_TILE
NEG_SLOPE = 0.2
NROW0 = 640
NROW15 = 400


def _lrelu(v):
    return jnp.where(v >= 0, v, NEG_SLOPE * v)


def _tc_prep(x, W, a_src, a_dst):
    def body(x_ref, w_ref, as_ref, ad_ref, h_ref, als_ref, ald_ref,
             sinit_ref, c_ref):
        h = lax.dot_general(
            x_ref[...], w_ref[...], (((1,), (0,)), ((), ())),
            precision=lax.Precision.HIGHEST,
            preferred_element_type=jnp.float32)
        h_ref[...] = h
        als = jnp.sum(h * as_ref[...][None, :], axis=1)
        ald = jnp.sum(h * ad_ref[...][None, :], axis=1)
        als_ref[...] = als
        ald_ref[...] = ald
        c = _lrelu(jnp.max(als) + jnp.max(ald))
        sinit_ref[...] = jnp.exp(_lrelu(als + ald) - c)
        c_ref[...] = jnp.full((LANES,), c, jnp.float32)

    return pl.pallas_call(
        body,
        out_shape=[
            jax.ShapeDtypeStruct((N, D), jnp.float32),
            jax.ShapeDtypeStruct((N,), jnp.float32),
            jax.ShapeDtypeStruct((N,), jnp.float32),
            jax.ShapeDtypeStruct((N,), jnp.float32),
            jax.ShapeDtypeStruct((LANES,), jnp.float32),
        ],
    )(x, W, a_src, a_dst)


def _sc_gat(src2d, dst2d, h, als, ald, sinit, cvec, bias):
    mesh = plsc.VectorSubcoreMesh(core_axis_name="c", subcore_axis_name="s")

    @functools.partial(
        pl.kernel,
        out_type=jax.ShapeDtypeStruct((N, D), jnp.float32),
        mesh=mesh,
        scratch_types=[
            pltpu.VMEM((N,), jnp.float32),
            pltpu.VMEM((N,), jnp.float32),
            pltpu.VMEM((N,), jnp.float32),
            pltpu.VMEM((ROWS_PER_TILE, 128), jnp.int32),
            pltpu.VMEM((ROWS_PER_TILE, 128), jnp.int32),
            pltpu.VMEM((ROWS_PER_TILE, 128), jnp.float32),
            pltpu.VMEM((128, DH), jnp.float32),
            pltpu.VMEM((128,), jnp.float32),
            pltpu.VMEM((LANES,), jnp.float32),
            pltpu.VMEM((128,), jnp.float32),
            pltpu.VMEM((NROW0,), jnp.float32),
            pltpu.VMEM_SHARED((N,), jnp.float32),
            pltpu.VMEM_SHARED((N, DH), jnp.float32),
            pltpu.VMEM_SHARED((N, DH), jnp.float32),
        ],
    )
    def k(src_hbm, dst_hbm, h_hbm, als_hbm, ald_hbm, sinit_hbm, c_hbm,
          bias_hbm, out_hbm, as_v, ad_v, rec_v, src_t, dst_t, w_t, rows_v,
          alpha_r, c_v, bias_v, sw_v, s_sh, h_sh, out_sh):
        cid = lax.axis_index("c")
        sid = lax.axis_index("s")

        pltpu.sync_copy(src_hbm.at[pl.ds(sid * ROWS_PER_TILE, ROWS_PER_TILE)],
                        src_t)
        pltpu.sync_copy(dst_hbm.at[pl.ds(sid * ROWS_PER_TILE, ROWS_PER_TILE)],
                        dst_t)
        pltpu.sync_copy(als_hbm, as_v)
        pltpu.sync_copy(ald_hbm, ad_v)
        pltpu.sync_copy(c_hbm, c_v)
        pltpu.sync_copy(bias_hbm, bias_v)

        @pl.loop(0, NROW0 // LANES)
        def _(i):
            sw_v[pl.ds(i * LANES, LANES)] = jnp.zeros((LANES,), jnp.float32)

        @pl.when(sid < 15)
        def _():
            pltpu.sync_copy(
                h_hbm.at[pl.ds(sid * NROW0, NROW0), pl.ds(cid * DH, DH)],
                h_sh.at[pl.ds(sid * NROW0, NROW0)])
            pltpu.sync_copy(sw_v, s_sh.at[pl.ds(sid * NROW0, NROW0)])

        @pl.when(sid == 15)
        def _():
            pltpu.sync_copy(
                h_hbm.at[pl.ds(15 * NROW0, NROW15), pl.ds(cid * DH, DH)],
                h_sh.at[pl.ds(15 * NROW0, NROW15)])
            pltpu.sync_copy(sw_v.at[pl.ds(0, NROW15)],
                            s_sh.at[pl.ds(15 * NROW0, NROW15)])

        plsc.subcore_barrier()

        @pl.loop(0, ROWS_PER_TILE)
        def _(j):
            grow = sid * ROWS_PER_TILE + j

            @pl.when(grow < EDGE_ROWS)
            def _():
                for kk in range(8):
                    sl = pl.ds(kk * LANES, LANES)
                    sv = src_t[j, sl]
                    dv = dst_t[j, sl]
                    es = plsc.load_gather(as_v, [sv])
                    ed = plsc.load_gather(ad_v, [dv])
                    e = _lrelu(es + ed)
                    w_t[j, sl] = jnp.exp(e - c_v[...])
                pltpu.sync_copy(w_t.at[j], s_sh.at[dst_t.at[j]], add=True)

        plsc.subcore_barrier()

        pltpu.sync_copy(s_sh, rec_v)
        pltpu.sync_copy(sinit_hbm, as_v)

        @pl.loop(0, N // LANES)
        def _(i):
            sl = pl.ds(i * LANES, LANES)
            rec_v[sl] = 1.0 / (rec_v[sl] + as_v[sl])

        def init_out(r0, sz):
            @pl.loop(0, sz // LANES)
            def _(i):
                sl = pl.ds(i * LANES, LANES)
                gsl = pl.ds(r0 + i * LANES, LANES)
                sw_v[sl] = as_v[gsl] * rec_v[gsl]

            @pl.loop(0, sz // 80)
            def _(q):
                q0 = r0 + q * 80
                pltpu.sync_copy(h_sh.at[pl.ds(q0, 80)],
                                rows_v.at[pl.ds(0, 80)])

                @pl.loop(0, 80)
                def _(r):
                    lr = q * 80 + r
                    splat = plsc.load_gather(
                        sw_v, [jnp.full((LANES,), lr, jnp.int32)])
                    for kk in range(4):
                        sl = pl.ds(kk * LANES, LANES)
                        bsl = pl.ds(cid * DH + kk * LANES, LANES)
                        rows_v[r, sl] = rows_v[r, sl] * splat + bias_v[bsl]

                pltpu.sync_copy(rows_v.at[pl.ds(0, 80)],
                                out_sh.at[pl.ds(q0, 80)])

        @pl.when(sid < 15)
        def _():
            init_out(sid * NROW0, NROW0)

        @pl.when(sid == 15)
        def _():
            init_out(15 * NROW0, NROW15)

        plsc.subcore_barrier()

        @pl.loop(0, ROWS_PER_TILE)
        def _(j):
            grow = sid * ROWS_PER_TILE + j

            @pl.when(grow < EDGE_ROWS)
            def _():
                for kk in range(8):
                    sl = pl.ds(kk * LANES, LANES)
                    dv = dst_t[j, sl]
                    r16 = plsc.load_gather(rec_v, [dv])
                    alpha_r[sl] = w_t[j, sl] * r16
                pltpu.sync_copy(h_sh.at[src_t.at[j]], rows_v)

                @pl.loop(0, 128)
                def _(r):
                    splat = plsc.load_gather(
                        alpha_r, [jnp.full((LANES,), r, jnp.int32)])
                    for kk in range(4):
                        sl = pl.ds(kk * LANES, LANES)
                        rows_v[r, sl] = rows_v[r, sl] * splat

                pltpu.sync_copy(rows_v, out_sh.at[dst_t.at[j]], add=True)

        plsc.subcore_barrier()

        @pl.when(sid < 15)
        def _():
            pltpu.sync_copy(
                out_sh.at[pl.ds(sid * NROW0, NROW0)],
                out_hbm.at[pl.ds(sid * NROW0, NROW0), pl.ds(cid * DH, DH)])

        @pl.when(sid == 15)
        def _():
            pltpu.sync_copy(
                out_sh.at[pl.ds(15 * NROW0, NROW15)],
                out_hbm.at[pl.ds(15 * NROW0, NROW15), pl.ds(cid * DH, DH)])

    return k(src2d, dst2d, h, als, ald, sinit, cvec, bias)


def kernel(x, edge_index, W, a_src, a_dst, bias):
    h, als, ald, sinit, cvec = _tc_prep(x, W, a_src, a_dst)
    pad = PAD_ROWS * 128 - E
    src2d = jnp.concatenate(
        [edge_index[0], jnp.zeros((pad,), jnp.int32)]).reshape(PAD_ROWS, 128)
    dst2d = jnp.concatenate(
        [edge_index[1], jnp.zeros((pad,), jnp.int32)]).reshape(PAD_ROWS, 128)
    return _sc_gat(src2d, dst2d, h, als, ald, sinit, cvec, bias)


# speedup vs baseline: 24.9174x; 24.9174x over previous
import dataclasses
import functools

import jax
import jax.numpy as jnp
from jax import lax
from jax.experimental import pallas as pl
from jax.experimental.pallas import tpu as pltpu
from jax.experimental.pallas import tpu_sc as plsc

N = 10000
E = 320000
D = 128
LANES = 16
EROW = 128
EDGE_ROWS = E // EROW
TROWS = 160
PAD_ROWS = 16 * TROWS
BROWS = 80
CH = 8
NEG_SLOPE = 0.2
NROW0 = 640
NROW15 = 400


def _lrelu(v):
    return jnp.where(v >= 0, v, NEG_SLOPE * v)


def _tc_prep(x, W, a_src, a_dst):
    def body(x_ref, w_ref, as_ref, ad_ref, h_ref, als_ref, ald_ref, c_ref):
        h = lax.dot_general(
            x_ref[...], w_ref[...], (((1,), (0,)), ((), ())),
            precision=lax.Precision.HIGHEST,
            preferred_element_type=jnp.float32)
        h_ref[...] = h
        als = jnp.sum(h * as_ref[...][None, :], axis=1)
        ald = jnp.sum(h * ad_ref[...][None, :], axis=1)
        als_ref[...] = als
        ald_ref[...] = ald
        c = _lrelu(jnp.max(als) + jnp.max(ald))
        c_ref[...] = jnp.full((LANES,), c, jnp.float32)

    return pl.pallas_call(
        body,
        out_shape=[
            jax.ShapeDtypeStruct((N, D), jnp.float32),
            jax.ShapeDtypeStruct((N,), jnp.float32),
            jax.ShapeDtypeStruct((N,), jnp.float32),
            jax.ShapeDtypeStruct((LANES,), jnp.float32),
        ],
    )(x, W, a_src, a_dst)


def _tc_finish(partials):
    def body(p_ref, o_ref):
        o_ref[...] = p_ref[0] + p_ref[1]

    return pl.pallas_call(
        body,
        out_shape=jax.ShapeDtypeStruct((N, D), jnp.float32),
    )(partials)


def _sc_gat(src2d, dst2d, h, als, ald, cvec, bias):
    mesh = plsc.VectorSubcoreMesh(core_axis_name="c", subcore_axis_name="s")
    cp = pltpu.CompilerParams()
    if "needs_layout_passes" in pltpu.CompilerParams.__dataclass_fields__:
        cp = dataclasses.replace(cp, needs_layout_passes=False)

    @functools.partial(
        pl.kernel,
        out_type=jax.ShapeDtypeStruct((2, N, D), jnp.float32),
        mesh=mesh,
        compiler_params=cp,
        scratch_types=[
            pltpu.VMEM((N,), jnp.float32),
            pltpu.VMEM((N,), jnp.float32),
            pltpu.VMEM((N,), jnp.float32),
            pltpu.VMEM((CH, EROW), jnp.int32),
            pltpu.VMEM((CH, EROW), jnp.int32),
            pltpu.VMEM((EROW,), jnp.float32),
            pltpu.VMEM((EROW, D), jnp.float32),
            pltpu.VMEM((EROW,), jnp.float32),
            pltpu.VMEM((LANES,), jnp.float32),
            pltpu.VMEM((D,), jnp.float32),
            pltpu.VMEM((NROW0,), jnp.float32),
            pltpu.VMEM_SHARED((N,), jnp.float32),
            pltpu.VMEM_SHARED((N, D), jnp.float32),
        ],
    )
    def k(src_hbm, dst_hbm, h_hbm, als_hbm, ald_hbm, c_hbm,
          bias_hbm, out_hbm, as_v, ad_v, rec_v, src_c, dst_c, wrow, rows_v,
          alpha_r, c_v, bias_v, sw_v, s_sh, out_sh):
        cid = lax.axis_index("c")
        sid = lax.axis_index("s")

        pltpu.sync_copy(als_hbm, as_v)
        pltpu.sync_copy(ald_hbm, ad_v)
        pltpu.sync_copy(c_hbm, c_v)
        pltpu.sync_copy(bias_hbm, bias_v)

        @pl.loop(0, NROW0 // LANES)
        def _(i):
            sw_v[pl.ds(i * LANES, LANES)] = jnp.zeros((LANES,), jnp.float32)

        @pl.when(sid < 15)
        def _():
            pltpu.sync_copy(sw_v, s_sh.at[pl.ds(sid * NROW0, NROW0)])

        @pl.when(sid == 15)
        def _():
            pltpu.sync_copy(sw_v.at[pl.ds(0, NROW15)],
                            s_sh.at[pl.ds(15 * NROW0, NROW15)])

        plsc.subcore_barrier()

        @pl.loop(0, TROWS // CH)
        def _(cb):
            row0 = sid * TROWS + cb * CH
            pltpu.sync_copy(src_hbm.at[pl.ds(row0, CH)], src_c)
            pltpu.sync_copy(dst_hbm.at[pl.ds(row0, CH)], dst_c)

            @pl.loop(0, CH)
            def _(j):
                @pl.when(row0 + j < EDGE_ROWS)
                def _():
                    for kk in range(EROW // LANES):
                        sl = pl.ds(kk * LANES, LANES)
                        sv = src_c[j, sl]
                        dv = dst_c[j, sl]
                        es = plsc.load_gather(as_v, [sv])
                        ed = plsc.load_gather(ad_v, [dv])
                        e = _lrelu(es + ed)
                        wrow[sl] = jnp.exp(e - c_v[...])
                    pltpu.sync_copy(wrow, s_sh.at[dst_c.at[j]], add=True)

        plsc.subcore_barrier()

        pltpu.sync_copy(s_sh, rec_v)

        @pl.loop(0, N // LANES)
        def _(i):
            sl = pl.ds(i * LANES, LANES)
            sinit = jnp.exp(_lrelu(as_v[sl] + ad_v[sl]) - c_v[...])
            rec_v[sl] = 1.0 / (rec_v[sl] + sinit)

        def init_out(r0, sz):
            @pl.when(cid == 0)
            def _():
                @pl.loop(0, sz // LANES)
                def _(i):
                    sl = pl.ds(i * LANES, LANES)
                    gsl = pl.ds(r0 + i * LANES, LANES)
                    sinit = jnp.exp(
                        _lrelu(as_v[gsl] + ad_v[gsl]) - c_v[...])
                    sw_v[sl] = sinit * rec_v[gsl]

            @pl.loop(0, sz // BROWS)
            def _(q):
                q0 = r0 + q * BROWS

                @pl.when(cid == 0)
                def _():
                    pltpu.sync_copy(h_hbm.at[pl.ds(q0, BROWS)],
                                    rows_v.at[pl.ds(0, BROWS)])

                    @pl.loop(0, BROWS)
                    def _(r):
                        lr = q * BROWS + r
                        splat = plsc.load_gather(
                            sw_v, [jnp.full((LANES,), lr, jnp.int32)])
                        for kk in range(D // LANES):
                            sl = pl.ds(kk * LANES, LANES)
                            rows_v[r, sl] = (rows_v[r, sl] * splat
                                             + bias_v[sl])

                @pl.when(cid == 1)
                def _():
                    @pl.loop(0, BROWS)
                    def _(r):
                        for kk in range(D // LANES):
                            sl = pl.ds(kk * LANES, LANES)
                            rows_v[r, sl] = jnp.zeros((LANES,), jnp.float32)

                pltpu.sync_copy(rows_v.at[pl.ds(0, BROWS)],
                                out_sh.at[pl.ds(q0, BROWS)])

        @pl.when(sid < 15)
        def _():
            init_out(sid * NROW0, NROW0)

        @pl.when(sid == 15)
        def _():
            init_out(15 * NROW0, NROW15)

        plsc.subcore_barrier()

        @pl.loop(0, BROWS // CH)
        def _(cb):
            row0 = sid * TROWS + cid * BROWS + cb * CH
            pltpu.sync_copy(src_hbm.at[pl.ds(row0, CH)], src_c)
            pltpu.sync_copy(dst_hbm.at[pl.ds(row0, CH)], dst_c)

            @pl.loop(0, CH)
            def _(j):
                @pl.when(row0 + j < EDGE_ROWS)
                def _():
                    for kk in range(EROW // LANES):
                        sl = pl.ds(kk * LANES, LANES)
                        sv = src_c[j, sl]
                        dv = dst_c[j, sl]
                        es = plsc.load_gather(as_v, [sv])
                        ed = plsc.load_gather(ad_v, [dv])
                        r16 = plsc.load_gather(rec_v, [dv])
                        e = _lrelu(es + ed)
                        alpha_r[sl] = jnp.exp(e - c_v[...]) * r16
                    pltpu.sync_copy(h_hbm.at[src_c.at[j]], rows_v)

                    @pl.loop(0, EROW)
                    def _(r):
                        splat = plsc.load_gather(
                            alpha_r, [jnp.full((LANES,), r, jnp.int32)])
                        for kk in range(D // LANES):
                            sl = pl.ds(kk * LANES, LANES)
                            rows_v[r, sl] = rows_v[r, sl] * splat

                    pltpu.sync_copy(rows_v, out_sh.at[dst_c.at[j]],
                                    add=True)

        plsc.subcore_barrier()

        @pl.when(sid < 15)
        def _():
            pltpu.sync_copy(
                out_sh.at[pl.ds(sid * NROW0, NROW0)],
                out_hbm.at[cid, pl.ds(sid * NROW0, NROW0)])

        @pl.when(sid == 15)
        def _():
            pltpu.sync_copy(
                out_sh.at[pl.ds(15 * NROW0, NROW15)],
                out_hbm.at[cid, pl.ds(15 * NROW0, NROW15)])

    return k(src2d, dst2d, h, als, ald, cvec, bias)


def kernel(x, edge_index, W, a_src, a_dst, bias):
    h, als, ald, cvec = _tc_prep(x, W, a_src, a_dst)
    pad = PAD_ROWS * EROW - E
    src2d = jnp.concatenate(
        [edge_index[0], jnp.zeros((pad,), jnp.int32)]).reshape(PAD_ROWS, EROW)
    dst2d = jnp.concatenate(
        [edge_index[1], jnp.zeros((pad,), jnp.int32)]).reshape(PAD_ROWS, EROW)
    partials = _sc_gat(src2d, dst2d, h, als, ald, cvec, bias)
    return _tc_finish(partials)


# speedup vs baseline: 31.2637x; 1.2547x over previous
import dataclasses
import functools

import jax
import jax.numpy as jnp
from jax import lax
from jax.experimental import pallas as pl
from jax.experimental.pallas import tpu as pltpu
from jax.experimental.pallas import tpu_sc as plsc

N = 10000
E = 320000
D = 128
LANES = 16
EROW = 128
EDGE_ROWS = E // EROW
TROWS = 160
PAD_ROWS = 16 * TROWS
BROWS = 80
CH = 8
NEG_SLOPE = 0.2
NROW0 = 640
NROW15 = 400
NPADS = 10240


def _lrelu(v):
    return jnp.where(v >= 0, v, NEG_SLOPE * v)


def _tc_prep(x, W, a_src, a_dst):
    def body(x_ref, w_ref, as_ref, ad_ref, h_ref, als_ref, ald_ref, c_ref):
        h = lax.dot_general(
            x_ref[...], w_ref[...], (((1,), (0,)), ((), ())),
            precision=lax.Precision.HIGHEST,
            preferred_element_type=jnp.float32)
        h_ref[...] = h
        als = jnp.sum(h * as_ref[...][None, :], axis=1)
        ald = jnp.sum(h * ad_ref[...][None, :], axis=1)
        als_ref[...] = als
        ald_ref[...] = ald
        c = _lrelu(jnp.max(als) + jnp.max(ald))
        c_ref[...] = jnp.full((LANES,), c, jnp.float32)

    return pl.pallas_call(
        body,
        out_shape=[
            jax.ShapeDtypeStruct((N, D), jnp.float32),
            jax.ShapeDtypeStruct((N,), jnp.float32),
            jax.ShapeDtypeStruct((N,), jnp.float32),
            jax.ShapeDtypeStruct((LANES,), jnp.float32),
        ],
    )(x, W, a_src, a_dst)


def _tc_finish(acc, s, h, als, ald, cvec, bias):
    def body(acc_ref, s_ref, h_ref, als_ref, ald_ref, c_ref, b_ref, o_ref):
        sinit = jnp.exp(_lrelu(als_ref[...] + ald_ref[...]) - c_ref[0])
        denom = s_ref[0] + s_ref[1] + sinit
        num = acc_ref[0] + acc_ref[1] + sinit[:, None] * h_ref[...]
        o_ref[...] = num / denom[:, None] + b_ref[...][None, :]

    return pl.pallas_call(
        body,
        out_shape=jax.ShapeDtypeStruct((N, D), jnp.float32),
    )(acc, s, h, als, ald, cvec, bias)


def _sc_gat(src2d, dst2d, h, als, ald, cvec):
    mesh = plsc.VectorSubcoreMesh(core_axis_name="c", subcore_axis_name="s")
    cp = pltpu.CompilerParams()
    if "needs_layout_passes" in pltpu.CompilerParams.__dataclass_fields__:
        cp = dataclasses.replace(cp, needs_layout_passes=False)

    @functools.partial(
        pl.kernel,
        out_type=[
            jax.ShapeDtypeStruct((2, N, D), jnp.float32),
            jax.ShapeDtypeStruct((2, NPADS), jnp.float32),
        ],
        mesh=mesh,
        compiler_params=cp,
        scratch_types=[
            pltpu.VMEM((N,), jnp.float32),
            pltpu.VMEM((N,), jnp.float32),
            pltpu.VMEM((CH, EROW), jnp.int32),
            pltpu.VMEM((CH, EROW), jnp.int32),
            pltpu.VMEM((EROW,), jnp.float32),
            pltpu.VMEM((EROW, D), jnp.float32),
            pltpu.VMEM((LANES,), jnp.float32),
            pltpu.VMEM_SHARED((NPADS,), jnp.float32),
            pltpu.VMEM_SHARED((N, D), jnp.float32),
        ],
    )
    def k(src_hbm, dst_hbm, h_hbm, als_hbm, ald_hbm, c_hbm,
          acc_hbm, s_hbm, as_v, ad_v, src_c, dst_c, wrow, rows_v,
          c_v, s_sh, out_sh):
        cid = lax.axis_index("c")
        sid = lax.axis_index("s")

        pltpu.sync_copy(als_hbm, as_v)
        pltpu.sync_copy(ald_hbm, ad_v)
        pltpu.sync_copy(c_hbm, c_v)

        @pl.loop(0, EROW)
        def _(r):
            for kk in range(D // LANES):
                rows_v[r, pl.ds(kk * LANES, LANES)] = (
                    jnp.zeros((LANES,), jnp.float32))
        for kk in range(EROW // LANES):
            wrow[pl.ds(kk * LANES, LANES)] = jnp.zeros((LANES,), jnp.float32)

        def zero_range(r0, sz):
            @pl.loop(0, sz // BROWS)
            def _(q):
                q0 = r0 + q * BROWS
                pltpu.sync_copy(rows_v.at[pl.ds(0, BROWS)],
                                out_sh.at[pl.ds(q0, BROWS)])
                pltpu.sync_copy(wrow.at[pl.ds(0, BROWS)],
                                s_sh.at[pl.ds(q0, BROWS)])

        @pl.when(sid < 15)
        def _():
            zero_range(sid * NROW0, NROW0)

        @pl.when(sid == 15)
        def _():
            zero_range(15 * NROW0, NROW15)

        plsc.subcore_barrier()

        @pl.loop(0, BROWS // CH)
        def _(cb):
            row0 = sid * TROWS + cid * BROWS + cb * CH
            pltpu.sync_copy(src_hbm.at[pl.ds(row0, CH)], src_c)
            pltpu.sync_copy(dst_hbm.at[pl.ds(row0, CH)], dst_c)

            @pl.loop(0, CH)
            def _(j):
                @pl.when(row0 + j < EDGE_ROWS)
                def _():
                    for kk in range(EROW // LANES):
                        sl = pl.ds(kk * LANES, LANES)
                        sv = src_c[j, sl]
                        dv = dst_c[j, sl]
                        es = plsc.load_gather(as_v, [sv])
                        ed = plsc.load_gather(ad_v, [dv])
                        e = _lrelu(es + ed)
                        wrow[sl] = jnp.exp(e - c_v[...])
                    pltpu.sync_copy(wrow, s_sh.at[dst_c.at[j]], add=True)
                    pltpu.sync_copy(h_hbm.at[src_c.at[j]], rows_v)

                    @pl.loop(0, EROW)
                    def _(r):
                        splat = plsc.load_gather(
                            wrow, [jnp.full((LANES,), r, jnp.int32)])
                        for kk in range(D // LANES):
                            sl = pl.ds(kk * LANES, LANES)
                            rows_v[r, sl] = rows_v[r, sl] * splat

                    pltpu.sync_copy(rows_v, out_sh.at[dst_c.at[j]],
                                    add=True)

        plsc.subcore_barrier()

        pltpu.sync_copy(s_sh.at[pl.ds(sid * NROW0, NROW0)],
                        s_hbm.at[cid, pl.ds(sid * NROW0, NROW0)])

        @pl.when(sid < 15)
        def _():
            pltpu.sync_copy(out_sh.at[pl.ds(sid * NROW0, NROW0)],
                            acc_hbm.at[cid, pl.ds(sid * NROW0, NROW0)])

        @pl.when(sid == 15)
        def _():
            pltpu.sync_copy(out_sh.at[pl.ds(15 * NROW0, NROW15)],
                            acc_hbm.at[cid, pl.ds(15 * NROW0, NROW15)])

    return k(src2d, dst2d, h, als, ald, cvec)


def kernel(x, edge_index, W, a_src, a_dst, bias):
    h, als, ald, cvec = _tc_prep(x, W, a_src, a_dst)
    pad = PAD_ROWS * EROW - E
    src2d = jnp.concatenate(
        [edge_index[0], jnp.zeros((pad,), jnp.int32)]).reshape(PAD_ROWS, EROW)
    dst2d = jnp.concatenate(
        [edge_index[1], jnp.zeros((pad,), jnp.int32)]).reshape(PAD_ROWS, EROW)
    acc, s = _sc_gat(src2d, dst2d, h, als, ald, cvec)
    return _tc_finish(acc, s[:, :N], h, als, ald, cvec, bias)


# speedup vs baseline: 34.2794x; 1.0965x over previous
import dataclasses
import functools

import jax
import jax.numpy as jnp
from jax import lax
from jax.experimental import pallas as pl
from jax.experimental.pallas import tpu as pltpu
from jax.experimental.pallas import tpu_sc as plsc

N = 10000
E = 320000
D = 128
LANES = 16
EROW = 128
EDGE_ROWS = E // EROW
TROWS = 160
PAD_ROWS = 16 * TROWS
BROWS = 80
CH = 8
NEG_SLOPE = 0.2
NROW0 = 640
NROW15 = 400
NPADS = 10240


def _lrelu(v):
    return jnp.where(v >= 0, v, NEG_SLOPE * v)


def _tc_prep(x, W, a_src, a_dst):
    def body(x_ref, w_ref, as_ref, ad_ref, h_ref, als_ref, ald_ref, c_ref):
        h = lax.dot_general(
            x_ref[...], w_ref[...], (((1,), (0,)), ((), ())),
            precision=lax.Precision.HIGHEST,
            preferred_element_type=jnp.float32)
        h_ref[...] = h
        als = jnp.sum(h * as_ref[...][None, :], axis=1)
        ald = jnp.sum(h * ad_ref[...][None, :], axis=1)
        als_ref[...] = als
        ald_ref[...] = ald
        c = _lrelu(jnp.max(als) + jnp.max(ald))
        c_ref[...] = jnp.full((LANES,), c, jnp.float32)

    return pl.pallas_call(
        body,
        out_shape=[
            jax.ShapeDtypeStruct((N, D), jnp.float32),
            jax.ShapeDtypeStruct((N,), jnp.float32),
            jax.ShapeDtypeStruct((N,), jnp.float32),
            jax.ShapeDtypeStruct((LANES,), jnp.float32),
        ],
    )(x, W, a_src, a_dst)


def _tc_finish(acc, s, h, als, ald, cvec, bias):
    def body(acc_ref, s_ref, h_ref, als_ref, ald_ref, c_ref, b_ref, o_ref):
        sinit = jnp.exp(_lrelu(als_ref[...] + ald_ref[...]) - c_ref[0])
        denom = s_ref[0] + s_ref[1] + sinit
        num = acc_ref[0] + acc_ref[1] + sinit[:, None] * h_ref[...]
        o_ref[...] = num / denom[:, None] + b_ref[...][None, :]

    return pl.pallas_call(
        body,
        out_shape=jax.ShapeDtypeStruct((N, D), jnp.float32),
    )(acc, s, h, als, ald, cvec, bias)


def _sc_gat(src2d, dst2d, h, als, ald, cvec):
    mesh = plsc.VectorSubcoreMesh(core_axis_name="c", subcore_axis_name="s")
    cp = pltpu.CompilerParams()
    if "needs_layout_passes" in pltpu.CompilerParams.__dataclass_fields__:
        cp = dataclasses.replace(cp, needs_layout_passes=False)

    @functools.partial(
        pl.kernel,
        out_type=[
            jax.ShapeDtypeStruct((2, N, D), jnp.float32),
            jax.ShapeDtypeStruct((2, NPADS), jnp.float32),
        ],
        mesh=mesh,
        compiler_params=cp,
        scratch_types=[
            pltpu.VMEM((N,), jnp.float32),
            pltpu.VMEM((N,), jnp.float32),
            pltpu.VMEM((CH, EROW), jnp.int32),
            pltpu.VMEM((CH, EROW), jnp.int32),
            pltpu.VMEM((EROW,), jnp.float32),
            pltpu.VMEM((EROW, D), jnp.float32),
            pltpu.VMEM((LANES,), jnp.float32),
            pltpu.VMEM_SHARED((NPADS,), jnp.float32),
            pltpu.VMEM_SHARED((N, D), jnp.float32),
            pltpu.SemaphoreType.DMA,
            pltpu.SemaphoreType.DMA,
        ],
    )
    def k(src_hbm, dst_hbm, h_hbm, als_hbm, ald_hbm, c_hbm,
          acc_hbm, s_hbm, as_v, ad_v, src_c, dst_c, wrow, rows_v,
          c_v, s_sh, out_sh, sem_a, sem_b):
        cid = lax.axis_index("c")
        sid = lax.axis_index("s")

        pltpu.sync_copy(als_hbm, as_v)
        pltpu.sync_copy(ald_hbm, ad_v)
        pltpu.sync_copy(c_hbm, c_v)

        @pl.loop(0, EROW)
        def _(r):
            for kk in range(D // LANES):
                rows_v[r, pl.ds(kk * LANES, LANES)] = (
                    jnp.zeros((LANES,), jnp.float32))
        for kk in range(EROW // LANES):
            wrow[pl.ds(kk * LANES, LANES)] = jnp.zeros((LANES,), jnp.float32)

        def zero_range(r0, sz):
            @pl.loop(0, sz // BROWS)
            def _(q):
                q0 = r0 + q * BROWS
                pltpu.sync_copy(rows_v.at[pl.ds(0, BROWS)],
                                out_sh.at[pl.ds(q0, BROWS)])
                pltpu.sync_copy(wrow.at[pl.ds(0, BROWS)],
                                s_sh.at[pl.ds(q0, BROWS)])

        @pl.when(sid < 15)
        def _():
            zero_range(sid * NROW0, NROW0)

        @pl.when(sid == 15)
        def _():
            zero_range(15 * NROW0, NROW15)

        plsc.subcore_barrier()

        @pl.loop(0, BROWS // CH)
        def _(cb):
            row0 = sid * TROWS + cid * BROWS + cb * CH
            pltpu.sync_copy(src_hbm.at[pl.ds(row0, CH)], src_c)
            pltpu.sync_copy(dst_hbm.at[pl.ds(row0, CH)], dst_c)

            @pl.loop(0, CH)
            def _(j):
                @pl.when(row0 + j < EDGE_ROWS)
                def _():
                    HALF = EROW // 2
                    ca = pltpu.async_copy(
                        h_hbm.at[src_c.at[j, pl.ds(0, HALF)]],
                        rows_v.at[pl.ds(0, HALF)], sem_a)
                    cb = pltpu.async_copy(
                        h_hbm.at[src_c.at[j, pl.ds(HALF, HALF)]],
                        rows_v.at[pl.ds(HALF, HALF)], sem_b)

                    for kk in range(EROW // LANES):
                        sl = pl.ds(kk * LANES, LANES)
                        sv = src_c[j, sl]
                        dv = dst_c[j, sl]
                        es = plsc.load_gather(as_v, [sv])
                        ed = plsc.load_gather(ad_v, [dv])
                        e = _lrelu(es + ed)
                        wrow[sl] = jnp.exp(e - c_v[...])
                    pltpu.sync_copy(wrow, s_sh.at[dst_c.at[j]], add=True)

                    def scale(r0):
                        @pl.loop(r0, r0 + HALF)
                        def _(r):
                            splat = plsc.load_gather(
                                wrow, [jnp.full((LANES,), r, jnp.int32)])
                            for kk in range(D // LANES):
                                sl = pl.ds(kk * LANES, LANES)
                                rows_v[r, sl] = rows_v[r, sl] * splat

                    ca.wait()
                    scale(0)
                    cb.wait()
                    scale(HALF)

                    pltpu.sync_copy(rows_v, out_sh.at[dst_c.at[j]],
                                    add=True)

        plsc.subcore_barrier()

        pltpu.sync_copy(s_sh.at[pl.ds(sid * NROW0, NROW0)],
                        s_hbm.at[cid, pl.ds(sid * NROW0, NROW0)])

        @pl.when(sid < 15)
        def _():
            pltpu.sync_copy(out_sh.at[pl.ds(sid * NROW0, NROW0)],
                            acc_hbm.at[cid, pl.ds(sid * NROW0, NROW0)])

        @pl.when(sid == 15)
        def _():
            pltpu.sync_copy(out_sh.at[pl.ds(15 * NROW0, NROW15)],
                            acc_hbm.at[cid, pl.ds(15 * NROW0, NROW15)])

    return k(src2d, dst2d, h, als, ald, cvec)


def kernel(x, edge_index, W, a_src, a_dst, bias):
    h, als, ald, cvec = _tc_prep(x, W, a_src, a_dst)
    pad = PAD_ROWS * EROW - E
    src2d = jnp.concatenate(
        [edge_index[0], jnp.zeros((pad,), jnp.int32)]).reshape(PAD_ROWS, EROW)
    dst2d = jnp.concatenate(
        [edge_index[1], jnp.zeros((pad,), jnp.int32)]).reshape(PAD_ROWS, EROW)
    acc, s = _sc_gat(src2d, dst2d, h, als, ald, cvec)
    return _tc_finish(acc, s[:, :N], h, als, ald, cvec, bias)
